# baseline (speedup 1.0000x reference)
import jax
import jax.numpy as jnp
from jax import lax
from jax.experimental import pallas as pl
from jax.experimental.pallas import tpu as pltpu
from jax.experimental.pallas import tpu_sc as plsc

IN_FEATURES = 3
CODE_NUM = 512
CODE_CHANNEL = 128
PT_NUM = 131072

NC = 2
NS = 16
NW = NC * NS
LANES = 16

PW = PT_NUM // NW
CH = 32
NCHUNK = PW // CH
RING = 4


def _sc_body(ct_hbm, t_hbm, d_hbm, out_hbm, cb, ib, fb, db, ab,
             csem, tsem, gsem, osem):
    wid = lax.axis_index("s") * NC + lax.axis_index("c")
    scale = (CODE_NUM - 1) / 2.0
    pbase = wid * PW

    def fire_coords(g, slot):
        for f in range(IN_FEATURES):
            pltpu.async_copy(
                ct_hbm.at[f, pl.ds(pbase + g * CH, CH)], cb.at[slot, f],
                csem.at[slot])

    def drain_coords(g, slot):
        for f in range(IN_FEATURES):
            pltpu.make_async_copy(
                ct_hbm.at[f, pl.ds(pbase + g * CH, CH)], cb.at[slot, f],
                csem.at[slot]).wait()

    def compute_idx(slot):
        for f in range(IN_FEATURES):
            for v in range(CH // LANES):
                s = pl.ds(v * LANES, LANES)
                c = (cb[slot, f, s] + 1.0) * scale
                ci = c.astype(jnp.int32)
                ci = jnp.minimum(jnp.maximum(ci, 0), CODE_NUM - 2)
                fb[slot, f, s] = c - ci.astype(jnp.float32)
                ib[slot, f, s] = ci + (f * CODE_NUM)

    def fire_t0(slot):
        pltpu.async_copy(t_hbm.at[ib.at[slot, 0]], ab.at[slot], tsem.at[slot])

    def drain_t0(slot):
        pltpu.make_async_copy(
            t_hbm.at[ib.at[slot, 0]], ab.at[slot], tsem.at[slot]).wait()

    def fire_t12_d(slot):
        for f in (1, 2):
            pltpu.async_copy(
                t_hbm.at[ib.at[slot, f]], ab.at[slot], gsem.at[slot], add=True)
        for f in range(IN_FEATURES):
            pltpu.async_copy(
                d_hbm.at[ib.at[slot, f]], db.at[slot, f], gsem.at[slot])

    def drain_t12_d(slot):
        for f in (1, 2):
            pltpu.make_async_copy(
                t_hbm.at[ib.at[slot, f]], ab.at[slot], gsem.at[slot]).wait()
        for f in range(IN_FEATURES):
            pltpu.make_async_copy(
                d_hbm.at[ib.at[slot, f]], db.at[slot, f], gsem.at[slot]).wait()

    def reduce_chunk(slot):
        @pl.loop(0, CH // LANES)
        def _grp(u):
            su = pl.ds(u * LANES, LANES)
            fv = [fb[slot, f, su] for f in range(IN_FEATURES)]
            for i2 in range(LANES):
                i = u * LANES + i2
                f0, f1, f2 = fv[0][i2], fv[1][i2], fv[2][i2]
                for v in range(CODE_CHANNEL // LANES):
                    lo = pl.ds(v * LANES, LANES)
                    for f, fr in ((0, f0), (1, f1), (2, f2)):
                        plsc.addupdate(ab.at[slot, i, lo],
                                       fr * db[slot, f, i, lo])

    def fire_out(g, slot):
        pltpu.async_copy(
            ab.at[slot], out_hbm.at[pl.ds(pbase + g * CH, CH)], osem.at[slot])

    def drain_out(g, slot):
        pltpu.make_async_copy(
            ab.at[slot], out_hbm.at[pl.ds(pbase + g * CH, CH)],
            osem.at[slot]).wait()

    fire_coords(0, 0)
    fire_coords(1, 1)
    fire_coords(2, 2)
    drain_coords(0, 0)
    compute_idx(0)
    fire_t0(0)
    drain_coords(1, 1)
    compute_idx(1)
    fire_t0(1)
    drain_t0(0)
    fire_t12_d(0)

    @pl.loop(0, NCHUNK)
    def _step(g):
        s0 = g % RING
        s1 = (g + 1) % RING
        s2 = (g + 2) % RING
        s3 = (g + 3) % RING

        @pl.when(g + 3 < NCHUNK)
        def _pf_coords():
            fire_coords(g + 3, s3)

        @pl.when(g + 2 < NCHUNK)
        def _stage_idx():
            drain_coords(g + 2, s2)
            compute_idx(s2)

            @pl.when(g >= 2)
            def _d_out():
                drain_out(g - 2, s2)

            fire_t0(s2)

        @pl.when(g + 1 < NCHUNK)
        def _stage_add():
            drain_t0(s1)
            fire_t12_d(s1)

        drain_t12_d(s0)
        reduce_chunk(s0)
        fire_out(g, s0)

    drain_out(NCHUNK - 2, (NCHUNK - 2) % RING)
    drain_out(NCHUNK - 1, (NCHUNK - 1) % RING)


@jax.jit
def _opt_pos_enc(ct, t, d):
    mesh = plsc.VectorSubcoreMesh(
        core_axis_name="c", subcore_axis_name="s", num_cores=NC, num_subcores=NS
    )
    return pl.kernel(
        _sc_body,
        out_type=jax.ShapeDtypeStruct((PT_NUM, CODE_CHANNEL), jnp.float32),
        mesh=mesh,
        scratch_types=[
            pltpu.VMEM((RING, IN_FEATURES, CH), jnp.float32),
            pltpu.VMEM((RING, IN_FEATURES, CH), jnp.int32),
            pltpu.VMEM((RING, IN_FEATURES, CH), jnp.float32),
            pltpu.VMEM((RING, IN_FEATURES, CH, CODE_CHANNEL), jnp.float32),
            pltpu.VMEM((RING, CH, CODE_CHANNEL), jnp.float32),
            pltpu.SemaphoreType.DMA((RING,)),
            pltpu.SemaphoreType.DMA((RING,)),
            pltpu.SemaphoreType.DMA((RING,)),
            pltpu.SemaphoreType.DMA((RING,)),
        ],
    )(ct, t, d)


def kernel(coords, shape_code):
    ct = coords[0].T
    t = shape_code.T
    t_shift = jnp.concatenate([t[1:], jnp.zeros((1, CODE_CHANNEL), t.dtype)])
    d = t_shift - t
    out = _opt_pos_enc(ct, t, d)
    return out[None]


# speedup vs baseline: 1.5148x; 1.5148x over previous
import jax
import jax.numpy as jnp
from jax import lax
from jax.experimental import pallas as pl
from jax.experimental.pallas import tpu as pltpu
from jax.experimental.pallas import tpu_sc as plsc

IN_FEATURES = 3
CODE_NUM = 512
CODE_CHANNEL = 128
PT_NUM = 131072

NC = 2
NS = 16
NW = NC * NS
LANES = 16

PW = PT_NUM // NW
CH = 32
NCHUNK = PW // CH
RING = 4


def _sc_body(ct_hbm, t_hbm, d_hbm, out_hbm, cb, ib, fb, db, ab, ob,
             csem, tsem, gsem, osem):
    wid = lax.axis_index("s") * NC + lax.axis_index("c")
    scale = (CODE_NUM - 1) / 2.0
    pbase = wid * PW

    def fire_coords(g, slot):
        for f in range(IN_FEATURES):
            pltpu.async_copy(
                ct_hbm.at[f, pl.ds(pbase + g * CH, CH)], cb.at[slot, f],
                csem.at[slot])

    def drain_coords(g, slot):
        for f in range(IN_FEATURES):
            pltpu.make_async_copy(
                ct_hbm.at[f, pl.ds(pbase + g * CH, CH)], cb.at[slot, f],
                csem.at[slot]).wait()

    def compute_idx(slot):
        for f in range(IN_FEATURES):
            for v in range(CH // LANES):
                s = pl.ds(v * LANES, LANES)
                c = (cb[slot, f, s] + 1.0) * scale
                ci = c.astype(jnp.int32)
                ci = jnp.minimum(jnp.maximum(ci, 0), CODE_NUM - 2)
                fb[slot, f, s] = c - ci.astype(jnp.float32)
                ib[slot, f, s] = ci + (f * CODE_NUM)

    def fire_t0(slot):
        pltpu.async_copy(t_hbm.at[ib.at[slot, 0]], ab.at[slot], tsem.at[slot])

    def drain_t0(slot):
        pltpu.make_async_copy(
            t_hbm.at[ib.at[slot, 0]], ab.at[slot], tsem.at[slot]).wait()

    def fire_t12_d(slot):
        for f in (1, 2):
            pltpu.async_copy(
                t_hbm.at[ib.at[slot, f]], ab.at[slot], gsem.at[slot], add=True)
        for f in range(IN_FEATURES):
            pltpu.async_copy(
                d_hbm.at[ib.at[slot, f]], db.at[slot, f], gsem.at[slot])

    def drain_t12_d(slot):
        for f in (1, 2):
            pltpu.make_async_copy(
                t_hbm.at[ib.at[slot, f]], ab.at[slot], gsem.at[slot]).wait()
        for f in range(IN_FEATURES):
            pltpu.make_async_copy(
                d_hbm.at[ib.at[slot, f]], db.at[slot, f], gsem.at[slot]).wait()

    def reduce_chunk(slot, oslot):
        @pl.loop(0, CH // LANES)
        def _grp(u):
            su = pl.ds(u * LANES, LANES)
            fv = [fb[slot, f, su] for f in range(IN_FEATURES)]
            for i2 in range(LANES):
                i = u * LANES + i2
                f0, f1, f2 = fv[0][i2], fv[1][i2], fv[2][i2]
                for v in range(CODE_CHANNEL // LANES):
                    lo = pl.ds(v * LANES, LANES)
                    acc = ab[slot, i, lo] + f0 * db[slot, 0, i, lo]
                    acc += f1 * db[slot, 1, i, lo] + f2 * db[slot, 2, i, lo]
                    ob[oslot, i, lo] = acc

    def fire_out(g, oslot):
        pltpu.async_copy(
            ob.at[oslot], out_hbm.at[pl.ds(pbase + g * CH, CH)],
            osem.at[oslot])

    def drain_out(g, oslot):
        pltpu.make_async_copy(
            ob.at[oslot], out_hbm.at[pl.ds(pbase + g * CH, CH)],
            osem.at[oslot]).wait()

    fire_coords(0, 0)
    fire_coords(1, 1)
    fire_coords(2, 2)
    drain_coords(0, 0)
    compute_idx(0)
    fire_t0(0)
    drain_coords(1, 1)
    compute_idx(1)
    fire_t0(1)
    drain_t0(0)
    fire_t12_d(0)

    @pl.loop(0, NCHUNK)
    def _step(g):
        s0 = g % RING
        s1 = (g + 1) % RING
        s2 = (g + 2) % RING
        s3 = (g + 3) % RING

        @pl.when(g + 3 < NCHUNK)
        def _pf_coords():
            fire_coords(g + 3, s3)

        @pl.when(g + 2 < NCHUNK)
        def _stage_idx():
            drain_coords(g + 2, s2)
            compute_idx(s2)

            fire_t0(s2)

        @pl.when(g + 1 < NCHUNK)
        def _stage_add():
            drain_t0(s1)
            fire_t12_d(s1)

        drain_t12_d(s0)

        @pl.when(g >= 2)
        def _d_out():
            drain_out(g - 2, g % 2)

        reduce_chunk(s0, g % 2)
        fire_out(g, g % 2)

    drain_out(NCHUNK - 2, NCHUNK % 2)
    drain_out(NCHUNK - 1, (NCHUNK - 1) % 2)


@jax.jit
def _opt_pos_enc(ct, t, d):
    mesh = plsc.VectorSubcoreMesh(
        core_axis_name="c", subcore_axis_name="s", num_cores=NC, num_subcores=NS
    )
    return pl.kernel(
        _sc_body,
        out_type=jax.ShapeDtypeStruct((PT_NUM, CODE_CHANNEL), jnp.float32),
        mesh=mesh,
        scratch_types=[
            pltpu.VMEM((RING, IN_FEATURES, CH), jnp.float32),
            pltpu.VMEM((RING, IN_FEATURES, CH), jnp.int32),
            pltpu.VMEM((RING, IN_FEATURES, CH), jnp.float32),
            pltpu.VMEM((RING, IN_FEATURES, CH, CODE_CHANNEL), jnp.float32),
            pltpu.VMEM((RING, CH, CODE_CHANNEL), jnp.float32),
            pltpu.VMEM((2, CH, CODE_CHANNEL), jnp.float32),
            pltpu.SemaphoreType.DMA((RING,)),
            pltpu.SemaphoreType.DMA((RING,)),
            pltpu.SemaphoreType.DMA((RING,)),
            pltpu.SemaphoreType.DMA((2,)),
        ],
    )(ct, t, d)


def kernel(coords, shape_code):
    ct = coords[0].T
    t = shape_code.T
    t_shift = jnp.concatenate([t[1:], jnp.zeros((1, CODE_CHANNEL), t.dtype)])
    d = t_shift - t
    out = _opt_pos_enc(ct, t, d)
    return out[None]


# speedup vs baseline: 1.5754x; 1.0400x over previous
import jax
import jax.numpy as jnp
from jax import lax
from jax.experimental import pallas as pl
from jax.experimental.pallas import tpu as pltpu
from jax.experimental.pallas import tpu_sc as plsc

IN_FEATURES = 3
CODE_NUM = 512
CODE_CHANNEL = 128
PT_NUM = 131072

NC = 2
NS = 16
NW = NC * NS
LANES = 16

PW = PT_NUM // NW
CH = 32
NCHUNK = PW // CH
RING = 4


def _sc_body(ct_hbm, t_hbm, d_hbm, out_hbm, cb, ib, fb, db, ab, ob,
             csem, tsem, gsem, osem):
    wid = lax.axis_index("s") * NC + lax.axis_index("c")
    scale = (CODE_NUM - 1) / 2.0
    pbase = wid * PW

    def fire_coords(g, slot):
        for f in range(IN_FEATURES):
            pltpu.async_copy(
                ct_hbm.at[f, pl.ds(pbase + g * CH, CH)], cb.at[slot, f],
                csem.at[slot])

    def drain_coords(g, slot):
        for f in range(IN_FEATURES):
            pltpu.make_async_copy(
                ct_hbm.at[f, pl.ds(pbase + g * CH, CH)], cb.at[slot, f],
                csem.at[slot]).wait()

    def compute_idx(slot):
        for f in range(IN_FEATURES):
            for v in range(CH // LANES):
                s = pl.ds(v * LANES, LANES)
                c = (cb[slot, f, s] + 1.0) * scale
                ci = c.astype(jnp.int32)
                ci = jnp.minimum(jnp.maximum(ci, 0), CODE_NUM - 2)
                fb[slot, f, s] = c - ci.astype(jnp.float32)
                ib[slot, f, s] = ci + (f * CODE_NUM)

    def fire_t0(slot):
        pltpu.async_copy(t_hbm.at[ib.at[slot, 0]], ab.at[slot], tsem.at[slot])

    def drain_t0(slot):
        pltpu.make_async_copy(
            t_hbm.at[ib.at[slot, 0]], ab.at[slot], tsem.at[slot]).wait()

    def fire_t12_d(slot):
        for f in (1, 2):
            pltpu.async_copy(
                t_hbm.at[ib.at[slot, f]], ab.at[slot], gsem.at[slot], add=True)
        for f in range(IN_FEATURES):
            pltpu.async_copy(
                d_hbm.at[ib.at[slot, f]], db.at[slot, f], gsem.at[slot])

    def drain_t12_d(slot):
        for f in (1, 2):
            pltpu.make_async_copy(
                t_hbm.at[ib.at[slot, f]], ab.at[slot], gsem.at[slot]).wait()
        for f in range(IN_FEATURES):
            pltpu.make_async_copy(
                d_hbm.at[ib.at[slot, f]], db.at[slot, f], gsem.at[slot]).wait()

    def reduce_chunk(slot, oslot):
        @pl.loop(0, CH // LANES)
        def _grp(u):
            su = pl.ds(u * LANES, LANES)
            fv = [fb[slot, f, su] for f in range(IN_FEATURES)]
            frs = {}

            def loads(i2, v):
                i = u * LANES + i2
                lo = pl.ds(v * LANES, LANES)
                return (ab[slot, i, lo], db[slot, 0, i, lo],
                        db[slot, 1, i, lo], db[slot, 2, i, lo])

            blocks = [(i2, v) for i2 in range(LANES)
                      for v in range(CODE_CHANNEL // LANES)]
            pending = loads(*blocks[0])
            for k, (i2, v) in enumerate(blocks):
                cur, pending = pending, (
                    loads(*blocks[k + 1]) if k + 1 < len(blocks) else None)
                if i2 not in frs:
                    frs[i2] = tuple(fv[f][i2] for f in range(IN_FEATURES))
                f0, f1, f2 = frs[i2]
                a, d0, d1, d2 = cur
                acc = (a + f0 * d0) + (f1 * d1 + f2 * d2)
                ob[oslot, u * LANES + i2, pl.ds(v * LANES, LANES)] = acc

    def fire_out(g, oslot):
        pltpu.async_copy(
            ob.at[oslot], out_hbm.at[pl.ds(pbase + g * CH, CH)],
            osem.at[oslot])

    def drain_out(g, oslot):
        pltpu.make_async_copy(
            ob.at[oslot], out_hbm.at[pl.ds(pbase + g * CH, CH)],
            osem.at[oslot]).wait()

    fire_coords(0, 0)
    fire_coords(1, 1)
    fire_coords(2, 2)
    drain_coords(0, 0)
    compute_idx(0)
    fire_t0(0)
    drain_coords(1, 1)
    compute_idx(1)
    fire_t0(1)
    drain_t0(0)
    fire_t12_d(0)

    @pl.loop(0, NCHUNK)
    def _step(g):
        s0 = g % RING
        s1 = (g + 1) % RING
        s2 = (g + 2) % RING
        s3 = (g + 3) % RING

        @pl.when(g + 3 < NCHUNK)
        def _pf_coords():
            fire_coords(g + 3, s3)

        @pl.when(g + 2 < NCHUNK)
        def _stage_idx():
            drain_coords(g + 2, s2)
            compute_idx(s2)

            fire_t0(s2)

        @pl.when(g + 1 < NCHUNK)
        def _stage_add():
            drain_t0(s1)
            fire_t12_d(s1)

        drain_t12_d(s0)

        @pl.when(g >= 2)
        def _d_out():
            drain_out(g - 2, g % 2)

        reduce_chunk(s0, g % 2)
        fire_out(g, g % 2)

    drain_out(NCHUNK - 2, NCHUNK % 2)
    drain_out(NCHUNK - 1, (NCHUNK - 1) % 2)


@jax.jit
def _opt_pos_enc(ct, t, d):
    mesh = plsc.VectorSubcoreMesh(
        core_axis_name="c", subcore_axis_name="s", num_cores=NC, num_subcores=NS
    )
    return pl.kernel(
        _sc_body,
        out_type=jax.ShapeDtypeStruct((PT_NUM, CODE_CHANNEL), jnp.float32),
        mesh=mesh,
        scratch_types=[
            pltpu.VMEM((RING, IN_FEATURES, CH), jnp.float32),
            pltpu.VMEM((RING, IN_FEATURES, CH), jnp.int32),
            pltpu.VMEM((RING, IN_FEATURES, CH), jnp.float32),
            pltpu.VMEM((RING, IN_FEATURES, CH, CODE_CHANNEL), jnp.float32),
            pltpu.VMEM((RING, CH, CODE_CHANNEL), jnp.float32),
            pltpu.VMEM((2, CH, CODE_CHANNEL), jnp.float32),
            pltpu.SemaphoreType.DMA((RING,)),
            pltpu.SemaphoreType.DMA((RING,)),
            pltpu.SemaphoreType.DMA((RING,)),
            pltpu.SemaphoreType.DMA((2,)),
        ],
    )(ct, t, d)


def kernel(coords, shape_code):
    ct = coords[0].T
    t = shape_code.T
    t_shift = jnp.concatenate([t[1:], jnp.zeros((1, CODE_CHANNEL), t.dtype)])
    d = t_shift - t
    out = _opt_pos_enc(ct, t, d)
    return out[None]


# speedup vs baseline: 1.5760x; 1.0004x over previous
import jax
import jax.numpy as jnp
from jax import lax
from jax.experimental import pallas as pl
from jax.experimental.pallas import tpu as pltpu
from jax.experimental.pallas import tpu_sc as plsc

IN_FEATURES = 3
CODE_NUM = 512
CODE_CHANNEL = 128
PT_NUM = 131072

NC = 2
NS = 16
NW = NC * NS
LANES = 16

PW = PT_NUM // NW
CH = 64
NCHUNK = PW // CH
RING = 3


def _sc_body(ct_hbm, t_hbm, d_hbm, out_hbm, cb, ib, fb, db, ab, ob,
             csem, tsem, gsem, osem):
    wid = lax.axis_index("s") * NC + lax.axis_index("c")
    scale = (CODE_NUM - 1) / 2.0
    pbase = wid * PW

    def fire_coords(g, slot):
        for f in range(IN_FEATURES):
            pltpu.async_copy(
                ct_hbm.at[f, pl.ds(pbase + g * CH, CH)], cb.at[slot, f],
                csem.at[slot])

    def drain_coords(g, slot):
        for f in range(IN_FEATURES):
            pltpu.make_async_copy(
                ct_hbm.at[f, pl.ds(pbase + g * CH, CH)], cb.at[slot, f],
                csem.at[slot]).wait()

    def compute_idx(slot):
        for f in range(IN_FEATURES):
            for v in range(CH // LANES):
                s = pl.ds(v * LANES, LANES)
                c = (cb[slot, f, s] + 1.0) * scale
                ci = c.astype(jnp.int32)
                ci = jnp.minimum(jnp.maximum(ci, 0), CODE_NUM - 2)
                fb[slot, f, s] = c - ci.astype(jnp.float32)
                ib[slot, f, s] = ci + (f * CODE_NUM)

    def fire_t0(slot):
        pltpu.async_copy(t_hbm.at[ib.at[slot, 0]], ab.at[slot], tsem.at[slot])

    def drain_t0(slot):
        pltpu.make_async_copy(
            t_hbm.at[ib.at[slot, 0]], ab.at[slot], tsem.at[slot]).wait()

    def fire_t12_d(slot):
        for f in (1, 2):
            pltpu.async_copy(
                t_hbm.at[ib.at[slot, f]], ab.at[slot], gsem.at[slot], add=True)
        for f in range(IN_FEATURES):
            pltpu.async_copy(
                d_hbm.at[ib.at[slot, f]], db.at[slot, f], gsem.at[slot])

    def drain_t12_d(slot):
        for f in (1, 2):
            pltpu.make_async_copy(
                t_hbm.at[ib.at[slot, f]], ab.at[slot], gsem.at[slot]).wait()
        for f in range(IN_FEATURES):
            pltpu.make_async_copy(
                d_hbm.at[ib.at[slot, f]], db.at[slot, f], gsem.at[slot]).wait()

    def reduce_chunk(slot, oslot):
        @pl.loop(0, CH // LANES)
        def _grp(u):
            su = pl.ds(u * LANES, LANES)
            fv = [fb[slot, f, su] for f in range(IN_FEATURES)]
            frs = {}

            def loads(i2, v):
                i = u * LANES + i2
                lo = pl.ds(v * LANES, LANES)
                return (ab[slot, i, lo], db[slot, 0, i, lo],
                        db[slot, 1, i, lo], db[slot, 2, i, lo])

            blocks = [(i2, v) for i2 in range(LANES)
                      for v in range(CODE_CHANNEL // LANES)]
            pending = loads(*blocks[0])
            for k, (i2, v) in enumerate(blocks):
                cur, pending = pending, (
                    loads(*blocks[k + 1]) if k + 1 < len(blocks) else None)
                if i2 not in frs:
                    frs[i2] = tuple(fv[f][i2] for f in range(IN_FEATURES))
                f0, f1, f2 = frs[i2]
                a, d0, d1, d2 = cur
                acc = (a + f0 * d0) + (f1 * d1 + f2 * d2)
                ob[oslot, u * LANES + i2, pl.ds(v * LANES, LANES)] = acc

    def fire_out(g, oslot):
        pltpu.async_copy(
            ob.at[oslot], out_hbm.at[pl.ds(pbase + g * CH, CH)],
            osem.at[oslot])

    def drain_out(g, oslot):
        pltpu.make_async_copy(
            ob.at[oslot], out_hbm.at[pl.ds(pbase + g * CH, CH)],
            osem.at[oslot]).wait()

    fire_coords(0, 0)
    fire_coords(1, 1)
    fire_coords(2, 2)
    drain_coords(0, 0)
    compute_idx(0)
    fire_t0(0)
    drain_coords(1, 1)
    compute_idx(1)
    fire_t0(1)
    drain_t0(0)
    fire_t12_d(0)

    @pl.loop(0, NCHUNK)
    def _step(g):
        s0 = g % RING
        s1 = (g + 1) % RING
        s2 = (g + 2) % RING
        s3 = (g + 3) % RING

        @pl.when(g + 3 < NCHUNK)
        def _pf_coords():
            fire_coords(g + 3, s3)

        @pl.when(g + 2 < NCHUNK)
        def _stage_idx():
            drain_coords(g + 2, s2)
            compute_idx(s2)

            fire_t0(s2)

        @pl.when(g + 1 < NCHUNK)
        def _stage_add():
            drain_t0(s1)
            fire_t12_d(s1)

        drain_t12_d(s0)

        @pl.when(g >= 2)
        def _d_out():
            drain_out(g - 2, g % 2)

        reduce_chunk(s0, g % 2)
        fire_out(g, g % 2)

    drain_out(NCHUNK - 2, NCHUNK % 2)
    drain_out(NCHUNK - 1, (NCHUNK - 1) % 2)


@jax.jit
def _opt_pos_enc(ct, t, d):
    mesh = plsc.VectorSubcoreMesh(
        core_axis_name="c", subcore_axis_name="s", num_cores=NC, num_subcores=NS
    )
    return pl.kernel(
        _sc_body,
        out_type=jax.ShapeDtypeStruct((PT_NUM, CODE_CHANNEL), jnp.float32),
        mesh=mesh,
        scratch_types=[
            pltpu.VMEM((RING, IN_FEATURES, CH), jnp.float32),
            pltpu.VMEM((RING, IN_FEATURES, CH), jnp.int32),
            pltpu.VMEM((RING, IN_FEATURES, CH), jnp.float32),
            pltpu.VMEM((RING, IN_FEATURES, CH, CODE_CHANNEL), jnp.float32),
            pltpu.VMEM((RING, CH, CODE_CHANNEL), jnp.float32),
            pltpu.VMEM((2, CH, CODE_CHANNEL), jnp.float32),
            pltpu.SemaphoreType.DMA((RING,)),
            pltpu.SemaphoreType.DMA((RING,)),
            pltpu.SemaphoreType.DMA((RING,)),
            pltpu.SemaphoreType.DMA((2,)),
        ],
    )(ct, t, d)


def kernel(coords, shape_code):
    ct = coords[0].T
    t = shape_code.T
    t_shift = jnp.concatenate([t[1:], jnp.zeros((1, CODE_CHANNEL), t.dtype)])
    d = t_shift - t
    out = _opt_pos_enc(ct, t, d)
    return out[None]
